# baseline (speedup 1.0000x reference)
import functools
import math

import jax
import jax.numpy as jnp
from jax import lax
from jax.experimental import pallas as pl
from jax.experimental.pallas import tpu as pltpu

_EXP2_SCALE = 8.0 * math.log2(math.e)

_BLOCK_B = 4


def _half_attn(q, k, v):
    qk = lax.dot_general(
        q, k,
        dimension_numbers=(((2,), (2,)), ((0,), (0,))),
        preferred_element_type=jnp.float32)
    m = jnp.max(qk, axis=-1, keepdims=True)
    xb = (qk - m).astype(jnp.bfloat16)
    p = jnp.exp2(xb * jnp.bfloat16(_EXP2_SCALE))
    denom = jnp.sum(p.astype(jnp.float32), axis=-1, keepdims=True)
    pv = lax.dot_general(
        p, v,
        dimension_numbers=(((2,), (1,)), ((0,), (0,))),
        preferred_element_type=jnp.float32)
    return (pv * (1.0 / denom)).astype(jnp.bfloat16)


def _sdpa_body(q_ref, k_ref, v_ref, o_ref,
               qb, kb, vb, ob, in_sem, out_sem, *, bb, d):
    i = pl.program_id(0) * bb

    pltpu.make_async_copy(q_ref.at[pl.ds(i, bb)], qb, in_sem.at[0]).start()
    pltpu.make_async_copy(k_ref.at[pl.ds(i, bb)], kb, in_sem.at[1]).start()
    pltpu.make_async_copy(v_ref.at[pl.ds(i, bb)], vb, in_sem.at[2]).start()
    pltpu.make_async_copy(q_ref.at[pl.ds(i, bb)], qb, in_sem.at[0]).wait()
    pltpu.make_async_copy(k_ref.at[pl.ds(i, bb)], kb, in_sem.at[1]).wait()
    pltpu.make_async_copy(v_ref.at[pl.ds(i, bb)], vb, in_sem.at[2]).wait()

    v = vb[...]
    oa = _half_attn(qb[..., 0:d], kb[..., 0:d], v)
    obv = _half_attn(qb[..., d:2 * d], kb[..., d:2 * d], v)

    ob[..., 0:d] = oa[..., 0:d]
    ob[..., d:2 * d] = obv[..., d:2 * d]

    pltpu.make_async_copy(ob, o_ref.at[pl.ds(i, bb)], out_sem).start()
    pltpu.make_async_copy(ob, o_ref.at[pl.ds(i, bb)], out_sem).wait()


def _attention(q2, k2, v2, *, interpret=False):
    Bh, S, D2 = q2.shape
    bb, d = _BLOCK_B, D2 // 2
    anyspec = pl.BlockSpec(memory_space=pl.ANY)
    return pl.pallas_call(
        functools.partial(_sdpa_body, bb=bb, d=d),
        out_shape=jax.ShapeDtypeStruct((Bh, S, D2), jnp.bfloat16),
        grid=(Bh // bb,),
        in_specs=[anyspec, anyspec, anyspec],
        out_specs=anyspec,
        scratch_shapes=[
            pltpu.VMEM((bb, S, D2), jnp.bfloat16),
            pltpu.VMEM((bb, S, D2), jnp.bfloat16),
            pltpu.VMEM((bb, S, D2), jnp.bfloat16),
            pltpu.VMEM((bb, S, D2), jnp.bfloat16),
            pltpu.SemaphoreType.DMA((3,)),
            pltpu.SemaphoreType.DMA,
        ],
        compiler_params=pltpu.CompilerParams(
            dimension_semantics=("arbitrary",),
            vmem_limit_bytes=22 * 1024 * 1024),
        interpret=interpret,
    )(q2, k2, v2)


def kernel(query, key, value):
    B, S, D = query.shape
    h = B // 2

    q = query.astype(jnp.bfloat16)
    k = key.astype(jnp.bfloat16)
    v = value.astype(jnp.bfloat16)
    q2 = jnp.concatenate([q[:h], q[h:]], axis=-1)
    k2 = jnp.concatenate([k[:h], k[h:]], axis=-1)
    v2 = jnp.concatenate([v[:h], v[h:]], axis=-1)

    y2 = _attention(q2, k2, v2)

    y = jnp.concatenate([y2[..., :D], y2[..., D:]], axis=0)
    return y.astype(jnp.float32)


# speedup vs baseline: 1.5186x; 1.5186x over previous
import math

import jax
import jax.numpy as jnp
from jax import lax
from jax.experimental import pallas as pl
from jax.experimental.pallas import tpu as pltpu

_EXP2_SCALE = 8.0 * math.log2(math.e)


def _sdpa_body(q_ref, k_ref, v_ref, o_ref, vext_ref):
    q = q_ref[...].astype(jnp.bfloat16)
    k = k_ref[...].astype(jnp.bfloat16)
    qk = lax.dot_general(
        q, k,
        dimension_numbers=(((2,), (2,)), ((0,), (0,))),
        preferred_element_type=jnp.float32)

    m = jnp.max(qk, axis=-1, keepdims=True)
    xb = (qk - m).astype(jnp.bfloat16)
    p = jnp.exp2(xb * jnp.bfloat16(_EXP2_SCALE))

    bb, s, d = q_ref.shape
    vext_ref[..., 0:d] = v_ref[...].astype(jnp.bfloat16)
    vext_ref[..., d:d + 1] = jnp.ones((bb, s, 1), jnp.bfloat16)

    pv = lax.dot_general(
        p, vext_ref[...],
        dimension_numbers=(((2,), (1,)), ((0,), (0,))),
        preferred_element_type=jnp.float32)

    denom = pv[..., d:d + 1]
    o_ref[...] = (pv[..., 0:d] * (1.0 / denom)).astype(jnp.bfloat16)


def kernel(query, key, value):
    B, S, D = query.shape
    block_b = 16
    grid = (B // block_b,)

    spec = pl.BlockSpec((block_b, S, D), lambda b: (b, 0, 0))
    y = pl.pallas_call(
        _sdpa_body,
        out_shape=jax.ShapeDtypeStruct((B, S, D), jnp.bfloat16),
        grid=grid,
        in_specs=[spec, spec, spec],
        out_specs=spec,
        scratch_shapes=[pltpu.VMEM((block_b, S, 128), jnp.bfloat16)],
        compiler_params=pltpu.CompilerParams(
            dimension_semantics=("parallel",)),
    )(query, key, value)
    return y.astype(jnp.float32)
